# baseline (speedup 1.0000x reference)
import math

import jax
import jax.numpy as jnp
import numpy as np
from jax.experimental import pallas as pl
from jax.sharding import PartitionSpec as P

N_FFT = 1024
HOP = 256
N_FREQ = N_FFT // 2 + 1
IM_OFF = 520
M_ROWS = 1040
F_C = 128


def _dft_mat_t():
    n = np.arange(N_FFT, dtype=np.float64)[None, :]
    k = np.arange(N_FREQ, dtype=np.float64)[:, None]
    ang = 2.0 * np.pi * n * k / N_FFT
    m = np.zeros((M_ROWS, N_FFT), dtype=np.float64)
    m[:N_FREQ] = np.cos(ang)
    m[IM_OFF : IM_OFF + N_FREQ] = -np.sin(ang)
    return m


_DFT_T_NP = _dft_mat_t()


def _stft_kernel(x_ref, mt_ref, re_ref, im_ref):
    c = pl.program_id(0)
    mt = mt_ref[...]
    for b in range(x_ref.shape[0]):
        seg = x_ref[pl.ds(b, 1), pl.ds(c * (F_C * HOP), (F_C + 3) * HOP)]
        seg_t = seg.reshape(F_C + 3, HOP).T
        frames_t = jnp.concatenate(
            [seg_t[:, 0:F_C], seg_t[:, 1 : F_C + 1],
             seg_t[:, 2 : F_C + 2], seg_t[:, 3 : F_C + 3]],
            axis=0,
        )
        r = jnp.dot(mt, frames_t, preferred_element_type=jnp.float32)
        re_ref[:, b, :] = r[0:N_FREQ, :]
        im_ref[:, b, :] = r[IM_OFF : IM_OFF + N_FREQ, :]


def _stft_planes(xp, mt, n_frames, n_chunks):
    b_loc, t_need = xp.shape
    return pl.pallas_call(
        _stft_kernel,
        grid=(n_chunks,),
        in_specs=[
            pl.BlockSpec((b_loc, t_need), lambda c: (0, 0)),
            pl.BlockSpec((M_ROWS, N_FFT), lambda c: (0, 0)),
        ],
        out_specs=[
            pl.BlockSpec((N_FREQ, b_loc, F_C), lambda c: (0, 0, c)),
            pl.BlockSpec((N_FREQ, b_loc, F_C), lambda c: (0, 0, c)),
        ],
        out_shape=[
            jax.ShapeDtypeStruct((N_FREQ, b_loc, n_frames), jnp.float32),
            jax.ShapeDtypeStruct((N_FREQ, b_loc, n_frames), jnp.float32),
        ],
    )(xp, mt)


@jax.jit
def kernel(x, window):
    B, T = x.shape
    pad = N_FFT // 2
    n_frames = 1 + T // HOP
    n_chunks = -(-n_frames // F_C)
    t_need = (n_chunks * F_C + 3) * HOP
    xp = jnp.pad(x, ((0, 0), (pad, t_need - T - pad)), mode="reflect")

    mt = window[None, :] * jnp.asarray(_DFT_T_NP, dtype=jnp.float32)

    n_dev = len(jax.devices())
    while B % n_dev:
        n_dev -= 1

    if n_dev > 1:
        mesh = jax.make_mesh((n_dev,), ("d",))
        xp = jax.reshard(xp, jax.sharding.NamedSharding(mesh, P("d", None)))
        mt = jax.reshard(mt, jax.sharding.NamedSharding(mesh, P(None, None)))
        re_t, im_t = jax.shard_map(
            lambda xl, m: _stft_planes(xl, m, n_frames, n_chunks),
            mesh=mesh,
            in_specs=(P("d", None), P(None, None)),
            out_specs=(P(None, "d", None), P(None, "d", None)),
            check_vma=False,
        )(xp, mt)
    else:
        re_t, im_t = _stft_planes(xp, mt, n_frames, n_chunks)

    return jnp.transpose(jax.lax.complex(re_t, im_t), (1, 2, 0))


# speedup vs baseline: 1.1272x; 1.1272x over previous
import math

import jax
import jax.numpy as jnp
import numpy as np
from jax.experimental import pallas as pl

N_FFT = 1024
HOP = 256
N_FREQ = N_FFT // 2 + 1
IM_OFF = 520
M_ROWS = 1040
F_C = 256


def _dft_mat_t():
    n = np.arange(N_FFT, dtype=np.float64)[None, :]
    k = np.arange(N_FREQ, dtype=np.float64)[:, None]
    ang = 2.0 * np.pi * n * k / N_FFT
    m = np.zeros((M_ROWS, N_FFT), dtype=np.float64)
    m[:N_FREQ] = np.cos(ang)
    m[IM_OFF : IM_OFF + N_FREQ] = -np.sin(ang)
    return m


_DFT_T_NP = _dft_mat_t()


def _stft_kernel(x_ref, mt_ref, re_ref, im_ref):
    c = pl.program_id(0)
    mt = mt_ref[...]
    for b in range(x_ref.shape[0]):
        seg = x_ref[pl.ds(b, 1), pl.ds(c * (F_C * HOP), (F_C + 3) * HOP)]
        seg_t = seg.reshape(F_C + 3, HOP).T
        frames_t = jnp.concatenate(
            [seg_t[:, 0:F_C], seg_t[:, 1 : F_C + 1],
             seg_t[:, 2 : F_C + 2], seg_t[:, 3 : F_C + 3]],
            axis=0,
        )
        r = jnp.dot(mt, frames_t, preferred_element_type=jnp.float32)
        re_ref[:, b, :] = r[0:N_FREQ, :]
        im_ref[:, b, :] = r[IM_OFF : IM_OFF + N_FREQ, :]


@jax.jit
def kernel(x, window):
    B, T = x.shape
    pad = N_FFT // 2
    n_frames = 1 + T // HOP
    n_chunks = -(-n_frames // F_C)
    t_need = (n_chunks * F_C + 3) * HOP
    xp = jnp.pad(x, ((0, 0), (pad, t_need - T - pad)), mode="reflect")

    mt = window[None, :] * jnp.asarray(_DFT_T_NP, dtype=jnp.float32)

    re_t, im_t = pl.pallas_call(
        _stft_kernel,
        grid=(n_chunks,),
        in_specs=[
            pl.BlockSpec((B, t_need), lambda c: (0, 0)),
            pl.BlockSpec((M_ROWS, N_FFT), lambda c: (0, 0)),
        ],
        out_specs=[
            pl.BlockSpec((N_FREQ, B, F_C), lambda c: (0, 0, c)),
            pl.BlockSpec((N_FREQ, B, F_C), lambda c: (0, 0, c)),
        ],
        out_shape=[
            jax.ShapeDtypeStruct((N_FREQ, B, n_frames), jnp.float32),
            jax.ShapeDtypeStruct((N_FREQ, B, n_frames), jnp.float32),
        ],
    )(xp, mt)

    return jnp.transpose(jax.lax.complex(re_t, im_t), (1, 2, 0))
